# Initial kernel scaffold; baseline (speedup 1.0000x reference)
import jax
import jax.numpy as jnp
from jax.experimental import pallas as pl


def kernel(target, context, negatives, W_target, W_context):
    raise NotImplementedError("write your pallas kernel here")


# speedup vs baseline: 4.4086x; 4.4086x over previous
import functools

import jax
import jax.numpy as jnp
from jax import lax
from jax.experimental import pallas as pl
from jax.experimental.pallas import tpu as pltpu
from jax.experimental.pallas import tpu_sc as plsc

VOCAB = 1000000
DIM = 64
BATCH = 16384
NEG = 20

NC = 2
NS = 16
L = 16
NW = NC * NS
BW = BATCH // NW
CHUNK = 64
NCHUNK = BW // CHUNK
NEGC = CHUNK * NEG
IDXW = 128
NGATH = NEGC // IDXW
NVREG = DIM // L


def _sc_body(tgt_hbm, ctx_hbm, neg_hbm, wt_hbm, wc_hbm,
             pos_hbm, negp_hbm,
             tgt_v, ctx_v, neg_v, t_rows, c_rows, n_rows,
             pos_out, neg_out, sem):
    wid = lax.axis_index("s") * NC + lax.axis_index("c")
    base = wid * BW

    def chunk_body(ch, carry):
        off = base + ch * CHUNK

        pltpu.sync_copy(tgt_hbm.at[pl.ds(off, CHUNK)], tgt_v)
        pltpu.sync_copy(ctx_hbm.at[pl.ds(off, CHUNK)], ctx_v)
        pltpu.sync_copy(neg_hbm.at[pl.ds(off * NEG, NEGC)], neg_v)

        cps = [pltpu.async_copy(wt_hbm.at[tgt_v], t_rows, sem),
               pltpu.async_copy(wc_hbm.at[ctx_v], c_rows, sem)]
        for g in range(NGATH):
            cps.append(pltpu.async_copy(
                wc_hbm.at[neg_v.at[pl.ds(g * IDXW, IDXW)]],
                n_rows.at[pl.ds(g * IDXW, IDXW)], sem))
        for cp in cps:
            cp.wait()

        def elem_body(e, c2):
            t = [t_rows[e, pl.ds(j * L, L)] for j in range(NVREG)]
            c = [c_rows[e, pl.ds(j * L, L)] for j in range(NVREG)]
            p = t[0] * c[0]
            for j in range(1, NVREG):
                p = p + t[j] * c[j]
            pos_out[e, :] = p
            for k in range(NEG):
                r = e * NEG + k
                a = t[0] * n_rows[r, pl.ds(0, L)]
                for j in range(1, NVREG):
                    a = a + t[j] * n_rows[r, pl.ds(j * L, L)]
                neg_out[r, :] = a
            return c2

        lax.fori_loop(0, CHUNK, elem_body, 0)

        pltpu.sync_copy(pos_out, pos_hbm.at[pl.ds(off, CHUNK)])
        pltpu.sync_copy(neg_out, negp_hbm.at[pl.ds(off * NEG, NEGC)])
        return carry

    lax.fori_loop(0, NCHUNK, chunk_body, 0)


@jax.jit
def _sc_gather_score(target, context, neg_flat, W_target, W_context):
    mesh = plsc.VectorSubcoreMesh(core_axis_name="c", subcore_axis_name="s")
    return pl.kernel(
        _sc_body,
        out_type=(jax.ShapeDtypeStruct((BATCH, L), jnp.float32),
                  jax.ShapeDtypeStruct((BATCH * NEG, L), jnp.float32)),
        mesh=mesh,
        scratch_types=[
            pltpu.VMEM((CHUNK,), jnp.int32),
            pltpu.VMEM((CHUNK,), jnp.int32),
            pltpu.VMEM((NEGC,), jnp.int32),
            pltpu.VMEM((CHUNK, DIM), jnp.float32),
            pltpu.VMEM((CHUNK, DIM), jnp.float32),
            pltpu.VMEM((NEGC, DIM), jnp.float32),
            pltpu.VMEM((CHUNK, L), jnp.float32),
            pltpu.VMEM((NEGC, L), jnp.float32),
            pltpu.SemaphoreType.DMA,
        ],
        compiler_params=pltpu.CompilerParams(use_tc_tiling_on_sc=False),
    )(target, context, neg_flat, W_target, W_context)


_NB = 16
_PB = BATCH // _NB
_NBROWS = BATCH * NEG // _NB


def _tc_loss_body(pos_ref, neg_ref, out_ref):
    i = pl.program_id(0)
    pos = jnp.sum(pos_ref[...], axis=1)
    pls = jnp.log(jax.nn.sigmoid(pos) + 1e-10)
    neg = jnp.sum(neg_ref[...], axis=1)
    nls = jnp.log(jax.nn.sigmoid(-neg) + 1e-10)
    s = jnp.sum(pls) + jnp.sum(nls)
    prev = jnp.where(i == 0, 0.0, out_ref[0, 0])
    tot = prev + s
    out_ref[0, 0] = jnp.where(i == _NB - 1, -tot / BATCH, tot)


@jax.jit
def _tc_loss(pos_partial, neg_partial):
    out = pl.pallas_call(
        _tc_loss_body,
        grid=(_NB,),
        in_specs=[
            pl.BlockSpec((_PB, L), lambda i: (i, 0)),
            pl.BlockSpec((_NBROWS, L), lambda i: (i, 0)),
        ],
        out_specs=pl.BlockSpec(memory_space=pltpu.SMEM),
        out_shape=jax.ShapeDtypeStruct((1, 1), jnp.float32),
    )(pos_partial, neg_partial)
    return out[0, 0]


def kernel(target, context, negatives, W_target, W_context):
    neg_flat = negatives.reshape(BATCH * NEG)
    pos_p, neg_p = _sc_gather_score(target, context, neg_flat,
                                    W_target, W_context)
    return _tc_loss(pos_p, neg_p)
